# baseline (speedup 1.0000x reference)
import functools

import jax
import jax.numpy as jnp
from jax import lax
from jax.experimental import pallas as pl
from jax.experimental.pallas import tpu as pltpu
from jax.experimental.pallas import tpu_sc as plsc

_LAYERS = 32
_HID = 2048
_TOK = 32768
_LANES = 16
_NW = 32
_TPW = _TOK // _NW
_CHUNK = 4
_NBUF = 6
_NCH = _TPW // _CHUNK
_IDS_PW = _NCH * 8
_HEAD = _NBUF - 1
_TAIL = ((_NCH - _HEAD - 1) % _NBUF) + 1
_NGRP = (_NCH - _HEAD - _TAIL) // _NBUF


def _sc_body(z_hbm, ids_hbm, delta_hbm, out_hbm, delta_v, ids_v, zbuf,
             sem_in, sem_out):
    cid = lax.axis_index("c")
    sid = lax.axis_index("s")
    wid = sid * 2 + cid
    base = wid * _TPW

    pltpu.sync_copy(delta_hbm, delta_v)
    pltpu.sync_copy(ids_hbm.at[pl.ds(wid * _IDS_PW, _IDS_PW)],
                    ids_v.at[pl.ds(0, _IDS_PW)])

    def start_in(c, b):
        pltpu.async_copy(
            z_hbm.at[pl.ds(base + c * _CHUNK, _CHUNK)], zbuf.at[b],
            sem_in.at[b])

    def wait_in(c, b):
        pltpu.make_async_copy(
            z_hbm.at[pl.ds(base + c * _CHUNK, _CHUNK)], zbuf.at[b],
            sem_in.at[b]).wait()

    def start_out(c, b):
        pltpu.async_copy(
            zbuf.at[b], out_hbm.at[pl.ds(base + c * _CHUNK, _CHUNK)],
            sem_out.at[b])

    def wait_out(c, b):
        pltpu.make_async_copy(
            zbuf.at[b], out_hbm.at[pl.ds(base + c * _CHUNK, _CHUNK)],
            sem_out.at[b]).wait()

    def compute(c, b):
        ids16 = ids_v[pl.ds(c * 8, _LANES)]
        offs = [ids16[t] * _HID for t in range(_CHUNK)]

        @plsc.parallel_loop(0, _HID, step=_LANES, unroll=8)
        def jbody(col):
            for t in range(_CHUNK):
                dv = delta_v[pl.ds(offs[t] + col, _LANES)]
                plsc.addupdate(zbuf.at[b, t, pl.ds(col, _LANES)], dv)

    def step(c, s, do_wait_out=True, do_start_in=True):
        s1 = (s + 1) % _NBUF
        if do_wait_out:
            wait_out(c - (_NBUF - 1), s1)
        if do_start_in:
            start_in(c + 1, s1)
        wait_in(c, s)
        compute(c, s)
        start_out(c, s)

    start_in(0, 0)
    for c in range(_HEAD):
        step(c, c % _NBUF, do_wait_out=(c >= _NBUF - 1))

    def group(g, carry):
        for i in range(_NBUF):
            c = _HEAD + g * _NBUF + i
            step(c, (_HEAD + i) % _NBUF)
        return carry

    lax.fori_loop(0, _NGRP, group, 0)

    for c in range(_NCH - _TAIL, _NCH):
        step(c, c % _NBUF, do_start_in=(c + 1 < _NCH))
    for c in range(_NCH - (_NBUF - 1), _NCH):
        wait_out(c, c % _NBUF)


@jax.jit
def kernel(z, layer_ids, delta):
    ids = layer_ids.astype(jnp.int32).reshape(_TOK // _CHUNK, _CHUNK)
    ids = jnp.pad(ids, ((0, 0), (0, 8 - _CHUNK))).reshape(-1)
    run = functools.partial(
        pl.kernel,
        out_type=jax.ShapeDtypeStruct((_TOK, _HID), jnp.float32),
        mesh=plsc.VectorSubcoreMesh(core_axis_name="c", subcore_axis_name="s"),
        compiler_params=pltpu.CompilerParams(needs_layout_passes=False),
        scratch_types=[
            pltpu.VMEM((_LAYERS * _HID,), jnp.float32),
            pltpu.VMEM((_IDS_PW + _LANES,), jnp.int32),
            pltpu.VMEM((_NBUF, _CHUNK, _HID), jnp.float32),
            pltpu.SemaphoreType.DMA((_NBUF,)),
            pltpu.SemaphoreType.DMA((_NBUF,)),
        ],
    )(_sc_body)
    return run(z, ids, delta.reshape(-1))


# speedup vs baseline: 1.2425x; 1.2425x over previous
import functools

import jax
import jax.numpy as jnp
from jax import lax
from jax.experimental import pallas as pl
from jax.experimental.pallas import tpu as pltpu
from jax.experimental.pallas import tpu_sc as plsc

_LAYERS = 32
_HID = 2048
_TOK = 32768
_LANES = 16
_NW = 32
_TPW = _TOK // _NW
_CHUNK = 8
_NBUF = 3
_NCH = _TPW // _CHUNK


def _sc_body(z_hbm, ids_hbm, delta_hbm, out_hbm, delta_v, ids_v, zbuf,
             sem_in, sem_out):
    cid = lax.axis_index("c")
    sid = lax.axis_index("s")
    wid = sid * 2 + cid
    base = wid * _TPW

    pltpu.sync_copy(delta_hbm, delta_v)
    pltpu.sync_copy(ids_hbm.at[pl.ds(base, _TPW)], ids_v.at[pl.ds(0, _TPW)])

    def start_in(c, b):
        pltpu.async_copy(
            z_hbm.at[pl.ds(base + c * _CHUNK, _CHUNK)], zbuf.at[b],
            sem_in.at[b])

    def wait_in(c, b):
        pltpu.make_async_copy(
            z_hbm.at[pl.ds(base + c * _CHUNK, _CHUNK)], zbuf.at[b],
            sem_in.at[b]).wait()

    def start_out(c, b):
        pltpu.async_copy(
            zbuf.at[b], out_hbm.at[pl.ds(base + c * _CHUNK, _CHUNK)],
            sem_out.at[b])

    def wait_out(c, b):
        pltpu.make_async_copy(
            zbuf.at[b], out_hbm.at[pl.ds(base + c * _CHUNK, _CHUNK)],
            sem_out.at[b]).wait()

    def compute(c, b):
        ids16 = ids_v[pl.ds(c * _CHUNK, _LANES)]
        offs = [ids16[t] * _HID for t in range(_CHUNK)]

        @plsc.parallel_loop(0, _HID, step=_LANES, unroll=8)
        def jbody(col):
            for t in range(_CHUNK):
                dv = delta_v[pl.ds(offs[t] + col, _LANES)]
                plsc.addupdate(zbuf.at[b, t, pl.ds(col, _LANES)], dv)

    def step(c, s, do_wait_out=True, do_start_in=True):
        s1 = (s + 1) % _NBUF
        if do_wait_out:
            wait_out(c - 2, s1)
        if do_start_in:
            start_in(c + 1, s1)
        wait_in(c, s)
        compute(c, s)
        start_out(c, s)

    start_in(0, 0)
    step(0, 0, do_wait_out=False)
    step(1, 1, do_wait_out=False)

    def group(g, carry):
        for b in range(_NBUF):
            c = 2 + g * _NBUF + b
            step(c, (2 + b) % _NBUF)
        return carry

    lax.fori_loop(0, (_NCH - 5) // _NBUF, group, 0)

    step(_NCH - 3, (_NCH - 3) % _NBUF)
    step(_NCH - 2, (_NCH - 2) % _NBUF)
    step(_NCH - 1, (_NCH - 1) % _NBUF, do_start_in=False)
    wait_out(_NCH - 2, (_NCH - 2) % _NBUF)
    wait_out(_NCH - 1, (_NCH - 1) % _NBUF)


@jax.jit
def kernel(z, layer_ids, delta):
    run = functools.partial(
        pl.kernel,
        out_type=jax.ShapeDtypeStruct((_TOK, _HID), jnp.float32),
        mesh=plsc.VectorSubcoreMesh(core_axis_name="c", subcore_axis_name="s"),
        compiler_params=pltpu.CompilerParams(needs_layout_passes=False),
        scratch_types=[
            pltpu.VMEM((_LAYERS * _HID,), jnp.float32),
            pltpu.VMEM((_TPW + _LANES,), jnp.int32),
            pltpu.VMEM((_NBUF, _CHUNK, _HID), jnp.float32),
            pltpu.SemaphoreType.DMA((_NBUF,)),
            pltpu.SemaphoreType.DMA((_NBUF,)),
        ],
    )(_sc_body)
    return run(z, layer_ids.astype(jnp.int32), delta.reshape(-1))
